# baseline (speedup 1.0000x reference)
import dataclasses
import functools

import jax
import jax.numpy as jnp
from jax import lax
from jax.experimental import pallas as pl
from jax.experimental.pallas import tpu as pltpu
from jax.experimental.pallas import tpu_sc as plsc

B = 16384
D = 64
L = 16
NC = 2
NS = 16
NW = NC * NS
BPW = B // NW
G = BPW // L

_mesh = plsc.VectorSubcoreMesh(core_axis_name="c", subcore_axis_name="s")

_cp = pltpu.CompilerParams()
if "needs_layout_passes" in pltpu.CompilerParams.__dataclass_fields__:
    _cp = dataclasses.replace(_cp, needs_layout_passes=False)
if "use_tc_tiling_on_sc" in pltpu.CompilerParams.__dataclass_fields__:
    _cp = dataclasses.replace(_cp, use_tc_tiling_on_sc=False)


def _sc_embed_dot(w, x0, x1):
    @functools.partial(
        pl.kernel,
        out_type=jax.ShapeDtypeStruct((B,), jnp.float32),
        mesh=_mesh,
        compiler_params=_cp,
        scratch_types=[
            pltpu.VMEM((BPW,), jnp.int32),
            pltpu.VMEM((BPW,), jnp.int32),
            pltpu.VMEM((BPW, D), jnp.float32),
            pltpu.VMEM((BPW, D), jnp.float32),
            pltpu.VMEM((BPW * L,), jnp.float32),
            pltpu.VMEM((BPW,), jnp.float32),
            pltpu.SemaphoreType.DMA,
            pltpu.SemaphoreType.DMA,
        ],
    )
    def k(w_hbm, x0_hbm, x1_hbm, out_hbm,
          idx0, idx1, rows_a, rows_b, pv, res, sem0, sem1):
        wid = lax.axis_index("s") * NC + lax.axis_index("c")
        base = wid * BPW
        pltpu.sync_copy(x0_hbm.at[pl.ds(base, BPW)], idx0)
        pltpu.sync_copy(x1_hbm.at[pl.ds(base, BPW)], idx1)
        ca = pltpu.async_copy(w_hbm.at[idx0], rows_a, sem0)
        cb = pltpu.async_copy(w_hbm.at[idx1], rows_b, sem1)
        ca.wait()
        cb.wait()

        @pl.loop(0, BPW)
        def _(r):
            a_r = rows_a.at[r]
            b_r = rows_b.at[r]
            acc = a_r[pl.ds(0, L)] * b_r[pl.ds(0, L)]
            for kk in range(1, D // L):
                acc = acc + a_r[pl.ds(kk * L, L)] * b_r[pl.ds(kk * L, L)]
            pv[pl.ds(r * L, L)] = acc

        lane = lax.iota(jnp.int32, L)

        @pl.loop(0, G)
        def _(g):
            idxv = g * (L * L) + lane * L
            tot = plsc.load_gather(pv, [idxv])
            for j in range(1, L):
                tot = tot + plsc.load_gather(pv, [idxv + j])
            res[pl.ds(g * L, L)] = 1.0 / (1.0 + jnp.exp(-tot))

        pltpu.sync_copy(res, out_hbm.at[pl.ds(base, BPW)])

    return k(w, x0, x1)


def kernel(x, W_g):
    x0 = jnp.asarray(x[:, 0], dtype=jnp.int32)
    x1 = jnp.asarray(x[:, 1], dtype=jnp.int32)
    out = _sc_embed_dot(W_g, x0, x1)
    return out.reshape(B, 1)
